# baseline (speedup 1.0000x reference)
import functools

import jax
import jax.numpy as jnp
from jax import lax
from jax.experimental import pallas as pl
from jax.experimental.pallas import tpu as pltpu
from jax.experimental.pallas import tpu_sc as plsc

N = 10000
E = 320000
H = 16
NC = 2
NS = 16
EPT = E // (NC * NS)
ROW_STRIDE = 624
ROWS_PT = 640
K = 2000
NCH = EPT // K

_mesh = plsc.VectorSubcoreMesh(
    core_axis_name="c", subcore_axis_name="s", num_cores=NC, num_subcores=NS
)

_SC_PARAMS = pltpu.CompilerParams(use_tc_tiling_on_sc=False)

_AGG_SCRATCH = [
    pltpu.VMEM((NCH, K), jnp.int32),
    pltpu.VMEM((NCH, K), jnp.int32),
    pltpu.VMEM((2, K, H), jnp.float32),
    pltpu.VMEM((ROWS_PT, H), jnp.float32),
    pltpu.VMEM((ROWS_PT, H), jnp.float32),
    pltpu.VMEM((ROWS_PT, H), jnp.float32),
    pltpu.VMEM((H, H), jnp.float32),
    pltpu.VMEM_SHARED((N, H), jnp.float32),
    pltpu.SemaphoreType.DMA,
    pltpu.SemaphoreType.DMA,
]

_out2 = [
    jax.ShapeDtypeStruct((N, H), jnp.float32),
    jax.ShapeDtypeStruct((N, H), jnp.float32),
]
_out4 = _out2 + _out2


def _phase_b_and_writeback(src_hbm, dst_hbm, p0_hbm, p1_hbm,
                           sidx, didx, rows, wb, gtab, acc, sem0, sem1, c, t):
    base = (c * NS + t) * EPT
    for i in range(NCH):
        pltpu.sync_copy(src_hbm.at[pl.ds(base + i * K, K)], sidx.at[i])
        pltpu.sync_copy(dst_hbm.at[pl.ds(base + i * K, K)], didx.at[i])
    plsc.subcore_barrier()

    sems = (sem0, sem1)
    pltpu.async_copy(gtab.at[sidx.at[0]], rows.at[0], sems[0])
    for i in range(NCH):
        if i + 1 < NCH:
            pltpu.async_copy(gtab.at[sidx.at[i + 1]], rows.at[(i + 1) % 2],
                             sems[(i + 1) % 2])
        pltpu.make_async_copy(gtab.at[sidx.at[i]], rows.at[i % 2],
                              sems[i % 2]).wait()
        pltpu.sync_copy(rows.at[i % 2], acc.at[didx.at[i]], add=True)

    plsc.subcore_barrier()
    r0 = t * ROW_STRIDE
    pltpu.sync_copy(acc.at[pl.ds(r0, ROWS_PT)], wb)

    @pl.when(c == 0)
    def _():
        pltpu.sync_copy(wb, p0_hbm.at[pl.ds(r0, ROWS_PT)])

    @pl.when(c == 1)
    def _():
        pltpu.sync_copy(wb, p1_hbm.at[pl.ds(r0, ROWS_PT)])


@functools.partial(
    pl.kernel, out_type=_out2, mesh=_mesh, scratch_types=_AGG_SCRATCH,
    compiler_params=_SC_PARAMS,
)
def _sc_aggregate(g_hbm, src_hbm, dst_hbm, zrows_hbm, p0_hbm, p1_hbm,
                  sidx, didx, rows, va, vb, vc, wv, acc, sem0, sem1):
    c = lax.axis_index("c")
    t = lax.axis_index("s")
    r0 = t * ROW_STRIDE
    pltpu.sync_copy(zrows_hbm, vb)
    pltpu.sync_copy(vb, acc.at[pl.ds(r0, ROWS_PT)])
    _phase_b_and_writeback(src_hbm, dst_hbm, p0_hbm, p1_hbm,
                           sidx, didx, rows, va, g_hbm, acc, sem0, sem1, c, t)


@functools.partial(
    pl.kernel, out_type=_out4, mesh=_mesh, scratch_types=_AGG_SCRATCH,
    compiler_params=_SC_PARAMS,
)
def _sc_layer(p0_hbm, p1_hbm, r_hbm, w_hbm, src_hbm, dst_hbm, zrows_hbm,
              q0_hbm, q1_hbm, g0_hbm, g1_hbm,
              sidx, didx, rows, va, vb, vc, wv, acc, sem0, sem1):
    c = lax.axis_index("c")
    t = lax.axis_index("s")
    r0 = t * ROW_STRIDE
    pltpu.sync_copy(zrows_hbm, va)
    pltpu.sync_copy(va, acc.at[pl.ds(r0, ROWS_PT)])
    pltpu.sync_copy(p0_hbm.at[pl.ds(r0, ROWS_PT)], va)
    pltpu.sync_copy(p1_hbm.at[pl.ds(r0, ROWS_PT)], vb)
    pltpu.sync_copy(r_hbm.at[pl.ds(r0, ROWS_PT)], vc)
    pltpu.sync_copy(w_hbm, wv)
    wregs = [wv[k] for k in range(H)]

    dnums = lax.GatherDimensionNumbers(
        offset_dims=(), collapsed_slice_dims=(0,), start_index_map=(0,))

    @plsc.parallel_loop(0, ROWS_PT, unroll=8)
    def _row(n):
        h = jnp.maximum(vc[n] * (va[n] + vb[n]), 0.0)
        parts = []
        for p in range(4):
            k = 4 * p
            bk = lax.gather(h, jnp.full((H, 1), k, jnp.int32), dnums, (1,),
                            mode=lax.GatherScatterMode.PROMISE_IN_BOUNDS)
            acc_p = bk * wregs[k]
            for k in range(4 * p + 1, 4 * p + 4):
                bk = lax.gather(h, jnp.full((H, 1), k, jnp.int32), dnums,
                                (1,),
                                mode=lax.GatherScatterMode.PROMISE_IN_BOUNDS)
                acc_p = acc_p + bk * wregs[k]
            parts.append(acc_p)
        out = (parts[0] + parts[1]) + (parts[2] + parts[3])
        va[n] = vc[n] * out

    @pl.when(c == 0)
    def _():
        pltpu.sync_copy(va, g0_hbm.at[pl.ds(r0, ROWS_PT)])

    @pl.when(c == 1)
    def _():
        pltpu.sync_copy(va, g1_hbm.at[pl.ds(r0, ROWS_PT)])

    @pl.when(c == 0)
    def _():
        _phase_b_and_writeback(src_hbm, dst_hbm, q0_hbm, q1_hbm,
                               sidx, didx, rows, vb, g0_hbm, acc,
                               sem0, sem1, c, t)

    @pl.when(c == 1)
    def _():
        _phase_b_and_writeback(src_hbm, dst_hbm, q0_hbm, q1_hbm,
                               sidx, didx, rows, vb, g1_hbm, acc,
                               sem0, sem1, c, t)


def _prep_body(x_ref, w_ref, d0_ref, d1_ref, r_ref, g_ref):
    deg = jnp.maximum(d0_ref[...] + d1_ref[...], 1.0)
    r = lax.rsqrt(deg)
    r_ref[...] = r
    g_ref[...] = r * jnp.dot(x_ref[...], w_ref[...],
                             preferred_element_type=jnp.float32)


def _final_body(p0_ref, p1_ref, r_ref, o_ref):
    z = r_ref[...] * (p0_ref[...] + p1_ref[...])
    col = lax.broadcasted_iota(jnp.int32, z.shape, 1)
    zm = jnp.where(col < 10, z, -jnp.inf)
    m = jnp.max(zm, axis=1, keepdims=True)
    s = jnp.sum(jnp.exp(zm - m), axis=1, keepdims=True)
    out = z - m - jnp.log(s)
    o_ref[...] = out[:, :10]


def kernel(x, edge_index, W_in, W_mid, W_out):
    f32 = jnp.float32
    src = edge_index[0].astype(jnp.int32)
    dst = edge_index[1].astype(jnp.int32)
    zrows = jnp.zeros((ROWS_PT, H), dtype=f32)
    ones_tab = jnp.ones((N, H), dtype=f32)
    w_out_pad = jnp.zeros((H, H), dtype=f32).at[:, :10].set(W_out)

    d0, d1 = _sc_aggregate(ones_tab, src, dst, zrows)

    prep = pl.pallas_call(
        _prep_body,
        out_shape=[jax.ShapeDtypeStruct((N, H), f32),
                   jax.ShapeDtypeStruct((N, H), f32)],
    )
    r_tab, g = prep(x, W_in, d0, d1)

    p0, p1 = _sc_aggregate(g, src, dst, zrows)
    for i in range(W_mid.shape[0]):
        p0, p1, _, _ = _sc_layer(p0, p1, r_tab, W_mid[i], src, dst, zrows)
    p0, p1, _, _ = _sc_layer(p0, p1, r_tab, w_out_pad, src, dst, zrows)

    final = pl.pallas_call(
        _final_body,
        out_shape=jax.ShapeDtypeStruct((N, 10), f32),
    )
    return final(p0, p1, r_tab)


# speedup vs baseline: 1.1498x; 1.1498x over previous
import functools

import jax
import jax.numpy as jnp
from jax import lax
from jax.experimental import pallas as pl
from jax.experimental.pallas import tpu as pltpu
from jax.experimental.pallas import tpu_sc as plsc

N = 10000
E = 320000
H = 16
NC = 2
NS = 16
EPT = E // (NC * NS)
ROW_STRIDE = 624
ROWS_PT = 640
K = 2000
NCH = EPT // K

_mesh = plsc.VectorSubcoreMesh(
    core_axis_name="c", subcore_axis_name="s", num_cores=NC, num_subcores=NS
)

_SC_PARAMS = pltpu.CompilerParams(use_tc_tiling_on_sc=False)

_AGG_SCRATCH = [
    pltpu.VMEM((NCH, K), jnp.int32),
    pltpu.VMEM((NCH, K), jnp.int32),
    pltpu.VMEM((2, K, H), jnp.float32),
    pltpu.VMEM((ROWS_PT, H), jnp.float32),
    pltpu.VMEM((ROWS_PT, H), jnp.float32),
    pltpu.VMEM((ROWS_PT, H), jnp.float32),
    pltpu.VMEM((H, H), jnp.float32),
    pltpu.VMEM_SHARED((N, H), jnp.float32),
    pltpu.SemaphoreType.DMA,
    pltpu.SemaphoreType.DMA,
    pltpu.SemaphoreType.DMA,
    pltpu.SemaphoreType.DMA,
]

_out2 = [
    jax.ShapeDtypeStruct((N, H), jnp.float32),
    jax.ShapeDtypeStruct((N, H), jnp.float32),
]
_out4 = _out2 + _out2


def _phase_b_and_writeback(src_hbm, dst_hbm, p0_hbm, p1_hbm, sidx, didx,
                           rows, wb, gtab, acc, sem0, sem1, sem2, sem3, c, t):
    base = (c * NS + t) * EPT
    for i in range(NCH):
        pltpu.sync_copy(src_hbm.at[pl.ds(base + i * K, K)], sidx.at[i])
        pltpu.sync_copy(dst_hbm.at[pl.ds(base + i * K, K)], didx.at[i])
    plsc.subcore_barrier()

    gsem = (sem0, sem1)
    ssem = (sem2, sem3)
    pltpu.async_copy(gtab.at[sidx.at[0]], rows.at[0], gsem[0])
    for i in range(NCH):
        if i + 1 < NCH:
            if i >= 1:
                pltpu.make_async_copy(rows.at[(i + 1) % 2],
                                      acc.at[didx.at[i - 1]],
                                      ssem[(i - 1) % 2]).wait()
            pltpu.async_copy(gtab.at[sidx.at[i + 1]], rows.at[(i + 1) % 2],
                             gsem[(i + 1) % 2])
        pltpu.make_async_copy(gtab.at[sidx.at[i]], rows.at[i % 2],
                              gsem[i % 2]).wait()
        pltpu.async_copy(rows.at[i % 2], acc.at[didx.at[i]], ssem[i % 2],
                         add=True)
    pltpu.make_async_copy(rows.at[(NCH - 2) % 2], acc.at[didx.at[NCH - 2]],
                          ssem[(NCH - 2) % 2]).wait()
    pltpu.make_async_copy(rows.at[(NCH - 1) % 2], acc.at[didx.at[NCH - 1]],
                          ssem[(NCH - 1) % 2]).wait()

    plsc.subcore_barrier()
    r0 = t * ROW_STRIDE
    pltpu.sync_copy(acc.at[pl.ds(r0, ROWS_PT)], wb)

    @pl.when(c == 0)
    def _():
        pltpu.sync_copy(wb, p0_hbm.at[pl.ds(r0, ROWS_PT)])

    @pl.when(c == 1)
    def _():
        pltpu.sync_copy(wb, p1_hbm.at[pl.ds(r0, ROWS_PT)])


@functools.partial(
    pl.kernel, out_type=_out2, mesh=_mesh, scratch_types=_AGG_SCRATCH,
    compiler_params=_SC_PARAMS,
)
def _sc_aggregate(g_hbm, src_hbm, dst_hbm, zrows_hbm, p0_hbm, p1_hbm,
                  sidx, didx, rows, va, vb, vc, wv, acc,
                  sem0, sem1, sem2, sem3):
    c = lax.axis_index("c")
    t = lax.axis_index("s")
    r0 = t * ROW_STRIDE
    pltpu.sync_copy(zrows_hbm, vb)
    pltpu.sync_copy(vb, acc.at[pl.ds(r0, ROWS_PT)])
    _phase_b_and_writeback(src_hbm, dst_hbm, p0_hbm, p1_hbm, sidx, didx,
                           rows, va, g_hbm, acc, sem0, sem1, sem2, sem3, c, t)


@functools.partial(
    pl.kernel, out_type=_out4, mesh=_mesh, scratch_types=_AGG_SCRATCH,
    compiler_params=_SC_PARAMS,
)
def _sc_layer(p0_hbm, p1_hbm, r_hbm, w_hbm, src_hbm, dst_hbm, zrows_hbm,
              q0_hbm, q1_hbm, g0_hbm, g1_hbm,
              sidx, didx, rows, va, vb, vc, wv, acc,
              sem0, sem1, sem2, sem3):
    c = lax.axis_index("c")
    t = lax.axis_index("s")
    r0 = t * ROW_STRIDE
    pltpu.sync_copy(zrows_hbm, va)
    pltpu.sync_copy(va, acc.at[pl.ds(r0, ROWS_PT)])
    pltpu.sync_copy(p0_hbm.at[pl.ds(r0, ROWS_PT)], va)
    pltpu.sync_copy(p1_hbm.at[pl.ds(r0, ROWS_PT)], vb)
    pltpu.sync_copy(r_hbm.at[pl.ds(r0, ROWS_PT)], vc)
    pltpu.sync_copy(w_hbm, wv)
    wregs = [wv[k] for k in range(H)]

    dnums = lax.GatherDimensionNumbers(
        offset_dims=(), collapsed_slice_dims=(0,), start_index_map=(0,))

    @plsc.parallel_loop(0, ROWS_PT, unroll=4)
    def _row(n):
        h = jnp.maximum(vc[n] * (va[n] + vb[n]), 0.0)
        parts = []
        for p in range(4):
            k = 4 * p
            bk = lax.gather(h, jnp.full((H, 1), k, jnp.int32), dnums, (1,),
                            mode=lax.GatherScatterMode.PROMISE_IN_BOUNDS)
            acc_p = bk * wregs[k]
            for k in range(4 * p + 1, 4 * p + 4):
                bk = lax.gather(h, jnp.full((H, 1), k, jnp.int32), dnums,
                                (1,),
                                mode=lax.GatherScatterMode.PROMISE_IN_BOUNDS)
                acc_p = acc_p + bk * wregs[k]
            parts.append(acc_p)
        out = (parts[0] + parts[1]) + (parts[2] + parts[3])
        va[n] = vc[n] * out

    @pl.when(c == 0)
    def _():
        pltpu.sync_copy(va, g0_hbm.at[pl.ds(r0, ROWS_PT)])

    @pl.when(c == 1)
    def _():
        pltpu.sync_copy(va, g1_hbm.at[pl.ds(r0, ROWS_PT)])

    @pl.when(c == 0)
    def _():
        _phase_b_and_writeback(src_hbm, dst_hbm, q0_hbm, q1_hbm, sidx, didx,
                               rows, vb, g0_hbm, acc, sem0, sem1, sem2, sem3,
                               c, t)

    @pl.when(c == 1)
    def _():
        _phase_b_and_writeback(src_hbm, dst_hbm, q0_hbm, q1_hbm, sidx, didx,
                               rows, vb, g1_hbm, acc, sem0, sem1, sem2, sem3,
                               c, t)


def _prep_body(x_ref, w_ref, d0_ref, d1_ref, r_ref, g_ref):
    deg = jnp.maximum(d0_ref[...] + d1_ref[...], 1.0)
    r = lax.rsqrt(deg)
    r_ref[...] = r
    g_ref[...] = r * jnp.dot(x_ref[...], w_ref[...],
                             preferred_element_type=jnp.float32)


def _final_body(p0_ref, p1_ref, r_ref, o_ref):
    z = r_ref[...] * (p0_ref[...] + p1_ref[...])
    col = lax.broadcasted_iota(jnp.int32, z.shape, 1)
    zm = jnp.where(col < 10, z, -jnp.inf)
    m = jnp.max(zm, axis=1, keepdims=True)
    s = jnp.sum(jnp.exp(zm - m), axis=1, keepdims=True)
    out = z - m - jnp.log(s)
    o_ref[...] = out[:, :10]


def kernel(x, edge_index, W_in, W_mid, W_out):
    f32 = jnp.float32
    src = edge_index[0].astype(jnp.int32)
    dst = edge_index[1].astype(jnp.int32)
    zrows = jnp.zeros((ROWS_PT, H), dtype=f32)
    ones_tab = jnp.ones((N, H), dtype=f32)
    w_out_pad = jnp.zeros((H, H), dtype=f32).at[:, :10].set(W_out)

    d0, d1 = _sc_aggregate(ones_tab, src, dst, zrows)

    prep = pl.pallas_call(
        _prep_body,
        out_shape=[jax.ShapeDtypeStruct((N, H), f32),
                   jax.ShapeDtypeStruct((N, H), f32)],
    )
    r_tab, g = prep(x, W_in, d0, d1)

    p0, p1 = _sc_aggregate(g, src, dst, zrows)
    for i in range(W_mid.shape[0]):
        p0, p1, _, _ = _sc_layer(p0, p1, r_tab, W_mid[i], src, dst, zrows)
    p0, p1, _, _ = _sc_layer(p0, p1, r_tab, w_out_pad, src, dst, zrows)

    final = pl.pallas_call(
        _final_body,
        out_shape=jax.ShapeDtypeStruct((N, 10), f32),
    )
    return final(p0, p1, r_tab)
